# baseline (speedup 1.0000x reference)
import functools

import jax
import jax.numpy as jnp
from jax import lax
from jax.experimental import pallas as pl
from jax.experimental.pallas import tpu as pltpu
from jax.experimental.pallas import tpu_sc as plsc

NV = 50000
NP = 50176
D = 64
DH = 16
NQ = 4
NL = 3
E = 800000
EP = 802816
ER = EP // 128
NSUB = 16
VPT = NP // NSUB
EPT = ER // NSUB
GRP = EPT // 8
WBC = 784
BLK = 512
NB = NP // BLK

_mesh = plsc.VectorSubcoreMesh(
    core_axis_name="c", subcore_axis_name="s", num_cores=2, num_subcores=16
)
_sc_params = pltpu.CompilerParams(use_tc_tiling_on_sc=False)

_f32 = jnp.float32
_i32 = jnp.int32


@functools.partial(
    pl.kernel,
    out_type=(
        jax.ShapeDtypeStruct((2 * NP, 16), _f32),
        jax.ShapeDtypeStruct((2 * NP, 16), _f32),
    ),
    mesh=_mesh,
    compiler_params=_sc_params,
    scratch_types=[
        pltpu.VMEM((8, 128), _i32),
        pltpu.VMEM((128, 16), _f32),
        pltpu.VMEM((1568, 16), _f32),
        pltpu.VMEM((WBC, 16), _f32),
        pltpu.VMEM_SHARED((NP, 16), _f32),
    ],
)
def _deg_kernel(eraw, deg_out, dinv_out, idxv, onesv, zbuf, dbuf, acc):
    c = lax.axis_index("c")
    s = lax.axis_index("s")
    one = jnp.ones((16,), _f32)
    z = jnp.zeros((16,), _f32)

    def fill_ones(i, _):
        onesv[i, 0:16] = one
        return 0

    lax.fori_loop(0, 128, fill_ones, 0)

    def fill_zero(i, _):
        zbuf[i, 0:16] = z
        return 0

    lax.fori_loop(0, 1568, fill_zero, 0)

    vbase = s * VPT
    pltpu.sync_copy(zbuf, acc.at[pl.ds(vbase, 1568)])
    pltpu.sync_copy(zbuf, acc.at[pl.ds(vbase + 1568, 1568)])
    plsc.subcore_barrier()

    row0 = c * ER + s * EPT

    def group(g, _):
        pltpu.sync_copy(eraw.at[pl.ds(row0 + g * 8, 8)], idxv)
        for j in range(8):
            pltpu.sync_copy(onesv, acc.at[idxv.at[j]], add=True)
        return 0

    lax.fori_loop(0, GRP, group, 0)
    plsc.subcore_barrier()
    pltpu.sync_copy(acc.at[pl.ds(vbase, VPT)], deg_out.at[pl.ds(c * NP + vbase, VPT)])
    for ch in range(VPT // WBC):
        base = vbase + ch * WBC
        pltpu.sync_copy(acc.at[pl.ds(base, WBC)], dbuf)

        def recip(i, _):
            dg = dbuf[i, 0:16]
            dv = jnp.where(dg > 0, 1.0 / jnp.maximum(dg, 1e-12), 0.0)
            dbuf[i, 0:16] = dv
            return 0

        lax.fori_loop(0, WBC, recip, 0)
        pltpu.sync_copy(dbuf, dinv_out.at[pl.ds(c * NP + base, WBC)])


def _make_sc_pass(swap: bool, scale_wb: bool):
    @functools.partial(
        pl.kernel,
        out_type=jax.ShapeDtypeStruct((NP, 8, DH), _f32),
        mesh=_mesh,
        compiler_params=_sc_params,
        scratch_types=[
            pltpu.VMEM((3, 8, 128), _i32),
            pltpu.VMEM((3, 8, 128), _i32),
            pltpu.VMEM((16, 128, DH), _f32),
            pltpu.VMEM((1568, DH), _f32),
            pltpu.VMEM_SHARED((NP, DH), _f32),
            pltpu.SemaphoreType.DMA,
            pltpu.SemaphoreType.DMA,
            pltpu.SemaphoreType.DMA,
            pltpu.SemaphoreType.DMA,
        ],
    )
    def _sc_pass(eoffq, eraw, tbl, dinv, out,
                 gidxv, sidxv, rbufs, zbuf, acc,
                 isem, gsem, ssem0, ssem1):
        c = lax.axis_index("c")
        s = lax.axis_index("s")
        gside = (1 - c) if swap else c
        sside = 1 - gside
        z = jnp.zeros((16,), _f32)

        def fill_zero(i, _):
            zbuf[i, 0:16] = z
            return 0

        lax.fori_loop(0, 1568, fill_zero, 0)
        vbase = s * VPT
        r0 = s * EPT

        for q in range(NQ):
            def stage_idx(g, slot):
                pltpu.async_copy(
                    eoffq.at[q, pl.ds(gside * ER + r0 + g * 8, 8)],
                    gidxv.at[slot], isem,
                )
                pltpu.async_copy(
                    eraw.at[pl.ds(sside * ER + r0 + g * 8, 8)],
                    sidxv.at[slot], isem,
                )

            def wait_idx(slot):
                pltpu.make_async_copy(
                    eoffq.at[q, pl.ds(r0, 8)], gidxv.at[slot], isem
                ).wait()
                pltpu.make_async_copy(
                    eraw.at[pl.ds(r0, 8)], sidxv.at[slot], isem
                ).wait()

            pltpu.sync_copy(zbuf, acc.at[pl.ds(vbase, 1568)])
            pltpu.sync_copy(zbuf, acc.at[pl.ds(vbase + 1568, 1568)])
            plsc.subcore_barrier()
            stage_idx(0, 0)

            def group(g, _):
                slot = lax.rem(g, 3)
                p = lax.rem(g, 2)
                wait_idx(slot)

                @pl.when(g >= 2)
                def _():
                    for sem_i, ssem in ((0, ssem0), (1, ssem1)):
                        @pl.when(p == sem_i)
                        def _():
                            for j in range(8):
                                pltpu.make_async_copy(
                                    rbufs.at[j], acc.at[sidxv.at[slot, 0]], ssem
                                ).wait()

                @pl.when(g < GRP - 1)
                def _():
                    stage_idx(g + 1, lax.rem(g + 1, 3))

                for j in range(8):
                    pltpu.async_copy(
                        tbl.at[gidxv.at[slot, j]], rbufs.at[p * 8 + j], gsem
                    )
                for j in range(8):
                    pltpu.make_async_copy(
                        tbl.at[gidxv.at[slot, 0]], rbufs.at[j], gsem
                    ).wait()
                for sem_i, ssem in ((0, ssem0), (1, ssem1)):
                    @pl.when(p == sem_i)
                    def _():
                        for j in range(8):
                            pltpu.async_copy(
                                rbufs.at[p * 8 + j], acc.at[sidxv.at[slot, j]],
                                ssem, add=True,
                            )
                return 0

            lax.fori_loop(0, GRP, group, 0)
            for ssem in (ssem0, ssem1):
                for j in range(8):
                    pltpu.make_async_copy(
                        rbufs.at[j], acc.at[sidxv.at[0, 0]], ssem
                    ).wait()
            plsc.subcore_barrier()

            for core_val in (0, 1):
                g_s = (1 - core_val) if swap else core_val
                s_s = 1 - g_s
                col = s_s * NQ + q

                @pl.when(c == core_val)
                def _():
                    if scale_wb:
                        CH = 112
                        for ch in range(VPT // CH):
                            base = vbase + ch * CH
                            pltpu.sync_copy(
                                acc.at[pl.ds(base, CH)],
                                rbufs.at[0, pl.ds(0, CH)],
                            )
                            pltpu.sync_copy(
                                dinv.at[pl.ds(s_s * NP + base, CH)],
                                rbufs.at[1, pl.ds(0, CH)],
                            )

                            def scale(i, _):
                                rbufs[0, i, 0:16] = (
                                    rbufs[0, i, 0:16] * rbufs[1, i, 0:16]
                                )
                                return 0

                            lax.fori_loop(0, CH, scale, 0)
                            pltpu.sync_copy(
                                rbufs.at[0, pl.ds(0, CH)],
                                out.at[pl.ds(base, CH), col],
                            )
                    else:
                        pltpu.sync_copy(
                            acc.at[pl.ds(vbase, VPT)],
                            out.at[pl.ds(vbase, VPT), col],
                        )
            if q < NQ - 1:
                plsc.subcore_barrier()

    return _sc_pass


_sc_pass_a = _make_sc_pass(swap=False, scale_wb=True)
_sc_pass_b = _make_sc_pass(swap=True, scale_wb=False)


def _leaky(x):
    return jnp.where(x >= 0, x, 0.2 * x)


def _prep_body(degu_ref, degi_ref, all0_ref, s_ref, xs_ref):
    du = degu_ref[...]
    di = degi_ref[...]
    su = jnp.where(du > 0, lax.rsqrt(jnp.maximum(du, 1e-12)), 0.0)
    si = jnp.where(di > 0, lax.rsqrt(jnp.maximum(di, 1e-12)), 0.0)
    s128 = jnp.concatenate(
        [jnp.broadcast_to(su, (BLK, D)), jnp.broadcast_to(si, (BLK, D))], axis=1
    )
    s_ref[...] = s128
    xs_ref[...] = all0_ref[...] * s128


_tc_prep = pl.pallas_call(
    _prep_body,
    grid=(NB,),
    in_specs=[
        pl.BlockSpec((BLK, 1), lambda b: (b, 0)),
        pl.BlockSpec((BLK, 1), lambda b: (b, 0)),
        pl.BlockSpec((BLK, 2 * D), lambda b: (b, 0)),
    ],
    out_specs=[
        pl.BlockSpec((BLK, 2 * D), lambda b: (b, 0)),
        pl.BlockSpec((BLK, 2 * D), lambda b: (b, 0)),
    ],
    out_shape=[
        jax.ShapeDtypeStruct((NP, 2 * D), _f32),
        jax.ShapeDtypeStruct((NP, 2 * D), _f32),
    ],
)


def _make_dense(last: bool):
    def body(all_ref, acc_ref, g_ref, s_ref, wgc_ref, bgc_ref,
             wbi_ref, bbi_ref, allo_ref, acco_ref, xs_ref):
        al = all_ref[...]
        s128 = s_ref[...]
        g = g_ref[...] * s128
        sum_e = _leaky(
            jnp.dot(g, wgc_ref[...], preferred_element_type=_f32)
            + bgc_ref[...] + al
        )
        bi = _leaky(
            jnp.dot(al * g, wbi_ref[...], preferred_element_type=_f32)
            + bbi_ref[...]
        )
        new = sum_e + bi
        nu = jnp.sqrt(jnp.sum(new[:, :D] * new[:, :D], axis=1, keepdims=True))
        ni = jnp.sqrt(jnp.sum(new[:, D:] * new[:, D:], axis=1, keepdims=True))
        den = jnp.concatenate(
            [jnp.broadcast_to(nu, (BLK, D)), jnp.broadcast_to(ni, (BLK, D))],
            axis=1,
        )
        new = new / jnp.maximum(den, 1e-12)
        allo_ref[...] = new
        acc_o = acc_ref[...] + new
        if last:
            acc_o = acc_o * 0.25
        acco_ref[...] = acc_o
        xs_ref[...] = new * s128

    return pl.pallas_call(
        body,
        grid=(NB,),
        in_specs=[
            pl.BlockSpec((BLK, 2 * D), lambda b: (b, 0)),
            pl.BlockSpec((BLK, 2 * D), lambda b: (b, 0)),
            pl.BlockSpec((BLK, 2 * D), lambda b: (b, 0)),
            pl.BlockSpec((BLK, 2 * D), lambda b: (b, 0)),
            pl.BlockSpec((2 * D, 2 * D), lambda b: (0, 0)),
            pl.BlockSpec((1, 2 * D), lambda b: (0, 0)),
            pl.BlockSpec((2 * D, 2 * D), lambda b: (0, 0)),
            pl.BlockSpec((1, 2 * D), lambda b: (0, 0)),
        ],
        out_specs=[
            pl.BlockSpec((BLK, 2 * D), lambda b: (b, 0)),
            pl.BlockSpec((BLK, 2 * D), lambda b: (b, 0)),
            pl.BlockSpec((BLK, 2 * D), lambda b: (b, 0)),
        ],
        out_shape=[
            jax.ShapeDtypeStruct((NP, 2 * D), _f32),
            jax.ShapeDtypeStruct((NP, 2 * D), _f32),
            jax.ShapeDtypeStruct((NP, 2 * D), _f32),
        ],
    )


_tc_dense = _make_dense(last=False)
_tc_dense_last = _make_dense(last=True)


def kernel(edge_index, u_table, i_table, W_gc, b_gc, W_bi, b_bi):
    users = edge_index[0]
    items = edge_index[1]
    pad_e = EP - E
    users_p = jnp.concatenate([users, jnp.full((pad_e,), NV, _i32)]).reshape(ER, 128)
    items_p = jnp.concatenate([items, jnp.full((pad_e,), NV, _i32)]).reshape(ER, 128)
    eraw = jnp.concatenate([users_p, items_p], axis=0)
    eoffq = jnp.stack(
        [
            jnp.concatenate([users_p * 8 + q, items_p * 8 + 4 + q], axis=0)
            for q in range(NQ)
        ],
        axis=0,
    )

    pad_v = NP - NV
    up = jnp.concatenate([u_table, jnp.zeros((pad_v, D), _f32)], axis=0)
    ip = jnp.concatenate([i_table, jnp.zeros((pad_v, D), _f32)], axis=0)
    all0 = jnp.concatenate([up, ip], axis=1)

    z64 = jnp.zeros((D, D), _f32)
    wgc2 = [
        jnp.block([[W_gc[l].T, z64], [z64, W_gc[l].T]]) for l in range(NL)
    ]
    wbi2 = [
        jnp.block([[W_bi[l].T, z64], [z64, W_bi[l].T]]) for l in range(NL)
    ]
    bgc2 = [jnp.concatenate([b_gc[l], b_gc[l]])[None] for l in range(NL)]
    bbi2 = [jnp.concatenate([b_bi[l], b_bi[l]])[None] for l in range(NL)]

    deg16, dinv16 = _deg_kernel(eraw)
    degu = deg16[:NP, :1]
    degi = deg16[NP:, :1]
    s128, xs = _tc_prep(degu, degi, all0)

    all_e = all0
    acc = all0
    for l in range(NL):
        y = _sc_pass_a(eoffq, eraw, xs.reshape(NP * 8, DH), dinv16)
        g = _sc_pass_b(eoffq, eraw, y.reshape(NP * 8, DH), dinv16)
        dense = _tc_dense_last if l == NL - 1 else _tc_dense
        all_e, acc, xs = dense(
            all_e, acc, g.reshape(NP, 2 * D), s128,
            wgc2[l], bgc2[l], wbi2[l], bbi2[l],
        )
    return acc[:NV, :D], acc[:NV, D:]


# speedup vs baseline: 1.1097x; 1.1097x over previous
import functools

import jax
import jax.numpy as jnp
from jax import lax
from jax.experimental import pallas as pl
from jax.experimental.pallas import tpu as pltpu
from jax.experimental.pallas import tpu_sc as plsc

NV = 50000
NP = 50176
D = 64
DH = 16
NQ = 4
NL = 3
E = 800000
EP = 802816
ER = EP // 128
NSUB = 16
VPT = NP // NSUB
EPT = ER // NSUB
GRP = EPT // 8
WBC = 784
BLK = 512
NB = NP // BLK

_mesh = plsc.VectorSubcoreMesh(
    core_axis_name="c", subcore_axis_name="s", num_cores=2, num_subcores=16
)
_sc_params = pltpu.CompilerParams(use_tc_tiling_on_sc=False)

_f32 = jnp.float32
_i32 = jnp.int32


@functools.partial(
    pl.kernel,
    out_type=(
        jax.ShapeDtypeStruct((2 * NP, 16), _f32),
        jax.ShapeDtypeStruct((2 * NP, 16), _f32),
    ),
    mesh=_mesh,
    compiler_params=_sc_params,
    scratch_types=[
        pltpu.VMEM((8, 128), _i32),
        pltpu.VMEM((128, 16), _f32),
        pltpu.VMEM((1568, 16), _f32),
        pltpu.VMEM((WBC, 16), _f32),
        pltpu.VMEM_SHARED((NP, 16), _f32),
    ],
)
def _deg_kernel(eraw, deg_out, dinv_out, idxv, onesv, zbuf, dbuf, acc):
    c = lax.axis_index("c")
    s = lax.axis_index("s")
    one = jnp.ones((16,), _f32)
    z = jnp.zeros((16,), _f32)

    def fill_ones(i, _):
        onesv[i, 0:16] = one
        return 0

    lax.fori_loop(0, 128, fill_ones, 0)

    def fill_zero(i, _):
        zbuf[i, 0:16] = z
        return 0

    lax.fori_loop(0, 1568, fill_zero, 0)

    vbase = s * VPT
    pltpu.sync_copy(zbuf, acc.at[pl.ds(vbase, 1568)])
    pltpu.sync_copy(zbuf, acc.at[pl.ds(vbase + 1568, 1568)])
    plsc.subcore_barrier()

    row0 = c * ER + s * EPT

    def group(g, _):
        pltpu.sync_copy(eraw.at[pl.ds(row0 + g * 8, 8)], idxv)
        for j in range(8):
            pltpu.sync_copy(onesv, acc.at[idxv.at[j]], add=True)
        return 0

    lax.fori_loop(0, GRP, group, 0)
    plsc.subcore_barrier()
    pltpu.sync_copy(acc.at[pl.ds(vbase, VPT)], deg_out.at[pl.ds(c * NP + vbase, VPT)])
    for ch in range(VPT // WBC):
        base = vbase + ch * WBC
        pltpu.sync_copy(acc.at[pl.ds(base, WBC)], dbuf)

        def recip(i, _):
            dg = dbuf[i, 0:16]
            dv = jnp.where(dg > 0, 1.0 / jnp.maximum(dg, 1e-12), 0.0)
            dbuf[i, 0:16] = dv
            return 0

        lax.fori_loop(0, WBC, recip, 0)
        pltpu.sync_copy(dbuf, dinv_out.at[pl.ds(c * NP + base, WBC)])


def _make_sc_pass(swap: bool, scale_wb: bool):
    @functools.partial(
        pl.kernel,
        out_type=jax.ShapeDtypeStruct((NP, 8 * DH), _f32),
        mesh=_mesh,
        compiler_params=_sc_params,
        scratch_types=[
            pltpu.VMEM((3, 8, 128), _i32),
            pltpu.VMEM((3, 8, 128), _i32),
            pltpu.VMEM((16, 128, DH), _f32),
            pltpu.VMEM((1568, DH), _f32),
            pltpu.VMEM_SHARED((NP, DH), _f32),
            pltpu.SemaphoreType.DMA,
            pltpu.SemaphoreType.DMA,
            pltpu.SemaphoreType.DMA,
            pltpu.SemaphoreType.DMA,
        ],
    )
    def _sc_pass(eoffq, eraw, tbl, dinv, out,
                 gidxv, sidxv, rbufs, zbuf, acc,
                 isem, gsem, ssem0, ssem1):
        c = lax.axis_index("c")
        s = lax.axis_index("s")
        gside = (1 - c) if swap else c
        sside = 1 - gside
        z = jnp.zeros((16,), _f32)

        def fill_zero(i, _):
            zbuf[i, 0:16] = z
            return 0

        lax.fori_loop(0, 1568, fill_zero, 0)
        vbase = s * VPT
        r0 = s * EPT

        for q in range(NQ):
            def stage_idx(g, slot):
                pltpu.async_copy(
                    eoffq.at[q, pl.ds(gside * ER + r0 + g * 8, 8)],
                    gidxv.at[slot], isem,
                )
                pltpu.async_copy(
                    eraw.at[pl.ds(sside * ER + r0 + g * 8, 8)],
                    sidxv.at[slot], isem,
                )

            def wait_idx(slot):
                pltpu.make_async_copy(
                    eoffq.at[q, pl.ds(r0, 8)], gidxv.at[slot], isem
                ).wait()
                pltpu.make_async_copy(
                    eraw.at[pl.ds(r0, 8)], sidxv.at[slot], isem
                ).wait()

            pltpu.sync_copy(zbuf, acc.at[pl.ds(vbase, 1568)])
            pltpu.sync_copy(zbuf, acc.at[pl.ds(vbase + 1568, 1568)])
            plsc.subcore_barrier()
            stage_idx(0, 0)

            def group(g, _):
                slot = lax.rem(g, 3)
                p = lax.rem(g, 2)
                wait_idx(slot)

                @pl.when(g >= 2)
                def _():
                    for sem_i, ssem in ((0, ssem0), (1, ssem1)):
                        @pl.when(p == sem_i)
                        def _():
                            for j in range(8):
                                pltpu.make_async_copy(
                                    rbufs.at[j], acc.at[sidxv.at[slot, 0]], ssem
                                ).wait()

                @pl.when(g < GRP - 1)
                def _():
                    stage_idx(g + 1, lax.rem(g + 1, 3))

                for j in range(8):
                    pltpu.async_copy(
                        tbl.at[gidxv.at[slot, j]], rbufs.at[p * 8 + j], gsem
                    )
                for j in range(8):
                    pltpu.make_async_copy(
                        tbl.at[gidxv.at[slot, 0]], rbufs.at[j], gsem
                    ).wait()
                for sem_i, ssem in ((0, ssem0), (1, ssem1)):
                    @pl.when(p == sem_i)
                    def _():
                        for j in range(8):
                            pltpu.async_copy(
                                rbufs.at[p * 8 + j], acc.at[sidxv.at[slot, j]],
                                ssem, add=True,
                            )
                return 0

            lax.fori_loop(0, GRP, group, 0)
            for ssem in (ssem0, ssem1):
                for j in range(8):
                    pltpu.make_async_copy(
                        rbufs.at[j], acc.at[sidxv.at[0, 0]], ssem
                    ).wait()
            plsc.subcore_barrier()

            for core_val in (0, 1):
                g_s = (1 - core_val) if swap else core_val
                s_s = 1 - g_s
                col = s_s * NQ + q

                @pl.when(c == core_val)
                def _():
                    if scale_wb:
                        CH = 112
                        for ch in range(VPT // CH):
                            base = vbase + ch * CH
                            pltpu.sync_copy(
                                acc.at[pl.ds(base, CH)],
                                rbufs.at[0, pl.ds(0, CH)],
                            )
                            pltpu.sync_copy(
                                dinv.at[pl.ds(s_s * NP + base, CH)],
                                rbufs.at[1, pl.ds(0, CH)],
                            )

                            def scale(i, _):
                                rbufs[0, i, 0:16] = (
                                    rbufs[0, i, 0:16] * rbufs[1, i, 0:16]
                                )
                                return 0

                            lax.fori_loop(0, CH, scale, 0)
                            pltpu.sync_copy(
                                rbufs.at[0, pl.ds(0, CH)],
                                out.at[pl.ds(base, CH), pl.ds(col * DH, DH)],
                            )
                    else:
                        pltpu.sync_copy(
                            acc.at[pl.ds(vbase, VPT)],
                            out.at[pl.ds(vbase, VPT), pl.ds(col * DH, DH)],
                        )
            if q < NQ - 1:
                plsc.subcore_barrier()

    return _sc_pass


_sc_pass_a = _make_sc_pass(swap=False, scale_wb=True)
_sc_pass_b = _make_sc_pass(swap=True, scale_wb=False)


def _leaky(x):
    return jnp.where(x >= 0, x, 0.2 * x)


def _prep_body(degu_ref, degi_ref, all0_ref, s_ref, xs_ref):
    du = degu_ref[...]
    di = degi_ref[...]
    su = jnp.where(du > 0, lax.rsqrt(jnp.maximum(du, 1e-12)), 0.0)
    si = jnp.where(di > 0, lax.rsqrt(jnp.maximum(di, 1e-12)), 0.0)
    s128 = jnp.concatenate(
        [jnp.broadcast_to(su, (BLK, D)), jnp.broadcast_to(si, (BLK, D))], axis=1
    )
    s_ref[...] = s128
    xs_ref[...] = all0_ref[...] * s128


_tc_prep = pl.pallas_call(
    _prep_body,
    grid=(NB,),
    in_specs=[
        pl.BlockSpec((BLK, 1), lambda b: (b, 0)),
        pl.BlockSpec((BLK, 1), lambda b: (b, 0)),
        pl.BlockSpec((BLK, 2 * D), lambda b: (b, 0)),
    ],
    out_specs=[
        pl.BlockSpec((BLK, 2 * D), lambda b: (b, 0)),
        pl.BlockSpec((BLK, 2 * D), lambda b: (b, 0)),
    ],
    out_shape=[
        jax.ShapeDtypeStruct((NP, 2 * D), _f32),
        jax.ShapeDtypeStruct((NP, 2 * D), _f32),
    ],
)


def _make_dense(last: bool):
    def body(all_ref, acc_ref, g_ref, s_ref, wgc_ref, bgc_ref,
             wbi_ref, bbi_ref, allo_ref, acco_ref, xs_ref):
        al = all_ref[...]
        s128 = s_ref[...]
        g = g_ref[...] * s128
        sum_e = _leaky(
            jnp.dot(g, wgc_ref[...], preferred_element_type=_f32)
            + bgc_ref[...] + al
        )
        bi = _leaky(
            jnp.dot(al * g, wbi_ref[...], preferred_element_type=_f32)
            + bbi_ref[...]
        )
        new = sum_e + bi
        nu = jnp.sqrt(jnp.sum(new[:, :D] * new[:, :D], axis=1, keepdims=True))
        ni = jnp.sqrt(jnp.sum(new[:, D:] * new[:, D:], axis=1, keepdims=True))
        den = jnp.concatenate(
            [jnp.broadcast_to(nu, (BLK, D)), jnp.broadcast_to(ni, (BLK, D))],
            axis=1,
        )
        new = new / jnp.maximum(den, 1e-12)
        allo_ref[...] = new
        acc_o = acc_ref[...] + new
        if last:
            acc_o = acc_o * 0.25
        acco_ref[...] = acc_o
        xs_ref[...] = new * s128

    return pl.pallas_call(
        body,
        grid=(NB,),
        in_specs=[
            pl.BlockSpec((BLK, 2 * D), lambda b: (b, 0)),
            pl.BlockSpec((BLK, 2 * D), lambda b: (b, 0)),
            pl.BlockSpec((BLK, 2 * D), lambda b: (b, 0)),
            pl.BlockSpec((BLK, 2 * D), lambda b: (b, 0)),
            pl.BlockSpec((2 * D, 2 * D), lambda b: (0, 0)),
            pl.BlockSpec((1, 2 * D), lambda b: (0, 0)),
            pl.BlockSpec((2 * D, 2 * D), lambda b: (0, 0)),
            pl.BlockSpec((1, 2 * D), lambda b: (0, 0)),
        ],
        out_specs=[
            pl.BlockSpec((BLK, 2 * D), lambda b: (b, 0)),
            pl.BlockSpec((BLK, 2 * D), lambda b: (b, 0)),
            pl.BlockSpec((BLK, 2 * D), lambda b: (b, 0)),
        ],
        out_shape=[
            jax.ShapeDtypeStruct((NP, 2 * D), _f32),
            jax.ShapeDtypeStruct((NP, 2 * D), _f32),
            jax.ShapeDtypeStruct((NP, 2 * D), _f32),
        ],
    )


_tc_dense = _make_dense(last=False)
_tc_dense_last = _make_dense(last=True)


def kernel(edge_index, u_table, i_table, W_gc, b_gc, W_bi, b_bi):
    users = edge_index[0]
    items = edge_index[1]
    pad_e = EP - E
    users_p = jnp.concatenate([users, jnp.full((pad_e,), NV, _i32)]).reshape(ER, 128)
    items_p = jnp.concatenate([items, jnp.full((pad_e,), NV, _i32)]).reshape(ER, 128)
    eraw = jnp.concatenate([users_p, items_p], axis=0)
    eoffq = jnp.stack(
        [
            jnp.concatenate([users_p * 8 + q, items_p * 8 + 4 + q], axis=0)
            for q in range(NQ)
        ],
        axis=0,
    )

    pad_v = NP - NV
    up = jnp.concatenate([u_table, jnp.zeros((pad_v, D), _f32)], axis=0)
    ip = jnp.concatenate([i_table, jnp.zeros((pad_v, D), _f32)], axis=0)
    all0 = jnp.concatenate([up, ip], axis=1)

    z64 = jnp.zeros((D, D), _f32)
    wgc2 = [
        jnp.block([[W_gc[l].T, z64], [z64, W_gc[l].T]]) for l in range(NL)
    ]
    wbi2 = [
        jnp.block([[W_bi[l].T, z64], [z64, W_bi[l].T]]) for l in range(NL)
    ]
    bgc2 = [jnp.concatenate([b_gc[l], b_gc[l]])[None] for l in range(NL)]
    bbi2 = [jnp.concatenate([b_bi[l], b_bi[l]])[None] for l in range(NL)]

    deg16, dinv16 = _deg_kernel(eraw)
    degu = deg16[:NP, :1]
    degi = deg16[NP:, :1]
    s128, xs = _tc_prep(degu, degi, all0)

    all_e = all0
    acc = all0
    for l in range(NL):
        y = _sc_pass_a(eoffq, eraw, xs.reshape(NP * 8, DH), dinv16)
        g = _sc_pass_b(eoffq, eraw, y.reshape(NP * 8, DH), dinv16)
        dense = _tc_dense_last if l == NL - 1 else _tc_dense
        all_e, acc, xs = dense(
            all_e, acc, g, s128,
            wgc2[l], bgc2[l], wbi2[l], bbi2[l],
        )
    return acc[:NV, :D], acc[:NV, D:]
